# baseline (speedup 1.0000x reference)
import functools

import jax
import jax.numpy as jnp
from jax import lax
from jax.experimental import pallas as pl
from jax.experimental.pallas import tpu as pltpu
from jax.experimental.pallas import tpu_sc as plsc

NC = 2
NS = 16
L = 16
NW = NC * NS

B = 16384
D = 64
CH = 384
HIT_CAP = 1536
CHIT_CAP = 256
NRING = 4
PAD = 128
UNROLL = 4


def _gather_pass(tab, tail_tab, ids_hbm, out_ref, ids_v, hitu, hitb, cu, cb,
                 staged, tail_staged, ext, sidx, sem_in, sem_out, wid,
                 n_rows):
    n_full = n_rows // CH
    tail = n_rows - n_full * CH
    tail_owner = n_full % NW
    lanes = lax.iota(jnp.int32, L)

    def _out_rows(idx_row):
        return out_ref.at[idx_row]

    def _dummy_rows():
        return out_ref.at[pl.ds(B, L)]

    nmine = (n_full - 1 - wid) // NW + 1
    nmine = jnp.maximum(nmine, 0)

    def fire_chunk(i, slot):
        base = pl.multiple_of((wid + i * NW) * CH, CH)
        pltpu.make_async_copy(
            tab.at[:, pl.ds(base, CH)], staged.at[slot], sem_in).start()

    for p in range(3):
        @pl.when(nmine > p)
        def _(p=p):
            fire_chunk(p, p)

    pltpu.sync_copy(ids_hbm, ids_v)

    def scan_body(i, off):
        for k in range(UNROLL):
            g = i * UNROLL + k
            v = ids_v[pl.ds(g * L, L)]
            m = ((v // CH) % NW) == wid
            n = plsc.all_reduce_population_count(m)
            plsc.store_compressed(hitu.at[pl.ds(off, L)], v, mask=m)
            plsc.store_compressed(hitb.at[pl.ds(off, L)], g * L + lanes,
                                  mask=m)
            off = jnp.minimum(off + n[0], HIT_CAP)
        return off

    nh = lax.fori_loop(0, B // L // UNROLL, scan_body, 0)
    ngrp = (nh + L - 1) // L

    def extract(src, base, width, gctr):
        def rescan(g, off):
            u = hitu[pl.ds(g * L, L)]
            b = hitb[pl.ds(g * L, L)]
            m = (u >= base) & (u < base + width) & (g * L + lanes < nh)
            n = plsc.all_reduce_population_count(m)
            plsc.store_compressed(cu.at[pl.ds(off, L)], u - base, mask=m)
            plsc.store_compressed(cb.at[pl.ds(off, L)], b, mask=m)
            return jnp.minimum(off + n[0], CHIT_CAP)

        nc = lax.fori_loop(0, ngrp, rescan, 0)

        def group_body(g, gctr):
            ring = gctr % NRING

            @pl.when(gctr >= NRING)
            def _():
                pltpu.make_async_copy(
                    ext.at[ring], _dummy_rows(), sem_out).wait()

            ul = cu[pl.ds(g * L, L)]
            bv = cb[pl.ds(g * L, L)]
            valid = g * L + lanes < nc
            ul = jnp.where(valid, ul, 0)
            bv = jnp.where(valid, bv, B + wid)
            sidx[ring, :] = bv
            for l in range(L):
                u_l = ul[l]
                for d16 in range(D // L):
                    dvec = d16 * L + lanes
                    uvec = jnp.full((L,), u_l, jnp.int32)
                    vals = plsc.load_gather(src, [dvec, uvec])
                    ext[ring, l, pl.ds(d16 * L, L)] = vals
            pltpu.make_async_copy(
                ext.at[ring], _out_rows(sidx.at[ring]), sem_out).start()
            return gctr + 1

        return lax.fori_loop(0, (nc + L - 1) // L, group_body, gctr)

    def chunk_loop(i, gctr):
        slot = i % 3
        base = pl.multiple_of((wid + i * NW) * CH, CH)
        pltpu.make_async_copy(
            tab.at[:, pl.ds(base, CH)], staged.at[slot], sem_in).wait()
        gctr = extract(staged.at[slot], base, CH, gctr)

        @pl.when(i + 3 < nmine)
        def _():
            fire_chunk(i + 3, slot)

        return gctr

    gctr = lax.fori_loop(0, nmine, chunk_loop, 0)

    gctr2 = gctr
    if tail:
        def tail_extract():
            t_base = n_full * CH
            pltpu.sync_copy(tail_tab, tail_staged)
            return extract(tail_staged, t_base, tail, gctr)

        gctr2 = lax.cond(wid == tail_owner, tail_extract, lambda: gctr)

    def drain(i, carry):
        pltpu.make_async_copy(ext.at[0], _dummy_rows(), sem_out).wait()
        return carry

    lax.fori_loop(0, jnp.minimum(gctr2, NRING), drain, 0)


@functools.lru_cache(maxsize=None)
def _make_stream_kernel(NU, NM):
    mesh = plsc.VectorSubcoreMesh(core_axis_name="c", subcore_axis_name="s")

    @functools.partial(
        pl.kernel,
        out_type=(jax.ShapeDtypeStruct((B + PAD, 128), jnp.float32),
                  jax.ShapeDtypeStruct((B + PAD, 128), jnp.float32)),
        mesh=mesh,
        scratch_types=[
            pltpu.VMEM((B,), jnp.int32),
            pltpu.VMEM((HIT_CAP + L,), jnp.int32),
            pltpu.VMEM((HIT_CAP + L,), jnp.int32),
            pltpu.VMEM((CHIT_CAP + L,), jnp.int32),
            pltpu.VMEM((CHIT_CAP + L,), jnp.int32),
            pltpu.VMEM((3, D, CH), jnp.float32),
            pltpu.VMEM((D, NU % CH), jnp.float32),
            pltpu.VMEM((D, NM % CH), jnp.float32),
            pltpu.VMEM((NRING, L, 128), jnp.float32),
            pltpu.VMEM((NRING, L), jnp.int32),
            pltpu.SemaphoreType.DMA,
            pltpu.SemaphoreType.DMA,
        ],
        compiler_params=pltpu.CompilerParams(
            needs_layout_passes=False, use_tc_tiling_on_sc=True),
    )
    def stream_kernel(uid_hbm, mid_hbm, ut_hbm, mt_hbm, ut_tail, mt_tail,
                      uvec_hbm, mvec_hbm,
                      ids_v, hitu, hitb, cu, cb, staged, ut_ts, mt_ts,
                      ext, sidx, sem_in, sem_out):
        wid = lax.axis_index("s") * NC + lax.axis_index("c")

        _gather_pass(ut_hbm, ut_tail, uid_hbm, uvec_hbm, ids_v, hitu, hitb,
                     cu, cb, staged, ut_ts, ext, sidx, sem_in, sem_out,
                     wid, NU)
        _gather_pass(mt_hbm, mt_tail, mid_hbm, mvec_hbm, ids_v, hitu, hitb,
                     cu, cb, staged, mt_ts, ext, sidx, sem_in, sem_out,
                     wid, NM)

    return stream_kernel


@functools.lru_cache(maxsize=None)
def _make_dot_kernel():
    mesh = plsc.VectorSubcoreMesh(core_axis_name="c", subcore_axis_name="s")
    b_per_w = B // NW
    ST = 128

    @functools.partial(
        pl.kernel,
        out_type=jax.ShapeDtypeStruct((B,), jnp.float32),
        mesh=mesh,
        scratch_types=[
            pltpu.VMEM((ST * 128,), jnp.float32),
            pltpu.VMEM((ST * 128,), jnp.float32),
            pltpu.VMEM((ST * 128,), jnp.float32),
            pltpu.VMEM((ST * 128,), jnp.float32),
            pltpu.VMEM((b_per_w,), jnp.float32),
            pltpu.SemaphoreType.DMA,
        ],
        compiler_params=pltpu.CompilerParams(
            needs_layout_passes=False, use_tc_tiling_on_sc=True),
    )
    def dot_kernel(uvec_hbm, mvec_hbm, out_hbm, su0, su1, sm0, sm1, outv,
                   sem):
        wid = lax.axis_index("s") * NC + lax.axis_index("c")
        base = wid * b_per_w
        lanes = lax.iota(jnp.int32, L)
        n_stages = b_per_w // ST
        su = (su0, su1)
        sm = (sm0, sm1)

        def fire(i, slot):
            pltpu.make_async_copy(
                uvec_hbm.at[pl.ds((base + i * ST) * 128, ST * 128)],
                su[slot], sem).start()
            pltpu.make_async_copy(
                mvec_hbm.at[pl.ds((base + i * ST) * 128, ST * 128)],
                sm[slot], sem).start()

        fire(0, 0)
        fire(1, 1)

        for i in range(n_stages):
            slot = i % 2
            pltpu.make_async_copy(
                uvec_hbm.at[pl.ds((base + i * ST) * 128, ST * 128)],
                su[slot], sem).wait()
            pltpu.make_async_copy(
                mvec_hbm.at[pl.ds((base + i * ST) * 128, ST * 128)],
                sm[slot], sem).wait()

            def group_body(g, carry2, i=i, slot=slot):
                fl = (g * L + lanes) * 128
                acc0 = jnp.zeros((L,), jnp.float32)
                acc1 = jnp.zeros((L,), jnp.float32)
                for d in range(D // 2):
                    uu0 = plsc.load_gather(su[slot], [fl + d])
                    mm0 = plsc.load_gather(sm[slot], [fl + d])
                    acc0 = acc0 + uu0 * mm0
                    uu1 = plsc.load_gather(su[slot], [fl + (d + D // 2)])
                    mm1 = plsc.load_gather(sm[slot], [fl + (d + D // 2)])
                    acc1 = acc1 + uu1 * mm1
                outv[pl.ds(i * ST + g * L, L)] = acc0 + acc1
                return carry2

            lax.fori_loop(0, ST // L, group_body, 0)
            if i + 2 < n_stages:
                fire(i + 2, slot)

        pltpu.sync_copy(outv, out_hbm.at[pl.ds(base, b_per_w)])

    return dot_kernel


def kernel(user_ids, movie_ids, user_table, movie_table):
    NU = user_table.shape[0]
    NM = movie_table.shape[0]
    uids = user_ids.astype(jnp.int32)
    mids = movie_ids.astype(jnp.int32)
    ut_tail = user_table[(NU // CH) * CH:].T
    mt_tail = movie_table[(NM // CH) * CH:].T
    k1 = _make_stream_kernel(NU, NM)
    uvec, mvec = k1(uids, mids, user_table.T, movie_table.T, ut_tail,
                    mt_tail)
    k2 = _make_dot_kernel()
    return k2(uvec.reshape(-1), mvec.reshape(-1))


# speedup vs baseline: 1.4569x; 1.4569x over previous
import functools

import jax
import jax.numpy as jnp
from jax import lax
from jax.experimental import pallas as pl
from jax.experimental.pallas import tpu as pltpu
from jax.experimental.pallas import tpu_sc as plsc

NC = 2
NS = 16
L = 16
NW = NC * NS

B = 16384
D = 64
CH = 512
HIT_CAP = 1536
CHIT_CAP = 256
NRING = 8
PAD = 128
UNROLL = 4


def _gather_pass(tab, tail_tab, ids_hbm, out_ref, ids_v, hitu, hitb, cu, cb,
                 staged, tail_staged, ext, sidx, sem_in, sem_out, wid,
                 n_rows):
    n_full = n_rows // CH
    tail = n_rows - n_full * CH
    tail_owner = n_full % NW
    lanes = lax.iota(jnp.int32, L)

    def _out_rows(idx_row):
        return out_ref.at[idx_row]

    def _dummy_rows():
        return out_ref.at[pl.ds(B, L)]

    nmine = (n_full - 1 - wid) // NW + 1
    nmine = jnp.maximum(nmine, 0)

    def fire_chunk(i, slot):
        base = pl.multiple_of((wid + i * NW) * CH, CH)
        pltpu.make_async_copy(
            tab.at[:, pl.ds(base, CH)], staged.at[slot], sem_in).start()

    @pl.when(nmine > 0)
    def _():
        fire_chunk(0, 0)

    @pl.when(nmine > 1)
    def _():
        fire_chunk(1, 1)

    pltpu.sync_copy(ids_hbm, ids_v)

    def scan_body(i, off):
        for k in range(UNROLL):
            g = i * UNROLL + k
            v = ids_v[pl.ds(g * L, L)]
            m = ((v // CH) % NW) == wid
            n = plsc.all_reduce_population_count(m)
            plsc.store_compressed(hitu.at[pl.ds(off, L)], v, mask=m)
            plsc.store_compressed(hitb.at[pl.ds(off, L)], g * L + lanes,
                                  mask=m)
            off = jnp.minimum(off + n[0], HIT_CAP)
        return off

    nh = lax.fori_loop(0, B // L // UNROLL, scan_body, 0)
    ngrp = (nh + L - 1) // L

    def extract(src, base, width, gctr):
        def rescan(g, off):
            u = hitu[pl.ds(g * L, L)]
            b = hitb[pl.ds(g * L, L)]
            m = (u >= base) & (u < base + width) & (g * L + lanes < nh)
            n = plsc.all_reduce_population_count(m)
            plsc.store_compressed(cu.at[pl.ds(off, L)], u - base, mask=m)
            plsc.store_compressed(cb.at[pl.ds(off, L)], b, mask=m)
            return jnp.minimum(off + n[0], CHIT_CAP)

        nc = lax.fori_loop(0, ngrp, rescan, 0)

        def group_body(g, gctr):
            ring = gctr % NRING

            @pl.when(gctr >= NRING)
            def _():
                pltpu.make_async_copy(
                    ext.at[ring], _dummy_rows(), sem_out).wait()

            ul = cu[pl.ds(g * L, L)]
            bv = cb[pl.ds(g * L, L)]
            valid = g * L + lanes < nc
            ul = jnp.where(valid, ul, 0)
            bv = jnp.where(valid, bv, B + wid)
            sidx[ring, :] = bv
            for l in range(L):
                u_l = ul[l]
                for d16 in range(D // L):
                    dvec = d16 * L + lanes
                    uvec = jnp.full((L,), u_l, jnp.int32)
                    vals = plsc.load_gather(src, [dvec, uvec])
                    ext[ring, l, pl.ds(d16 * L, L)] = vals
            pltpu.make_async_copy(
                ext.at[ring], _out_rows(sidx.at[ring]), sem_out).start()
            return gctr + 1

        return lax.fori_loop(0, (nc + L - 1) // L, group_body, gctr)

    def chunk_loop(i, gctr):
        slot = i % 2
        base = pl.multiple_of((wid + i * NW) * CH, CH)
        pltpu.make_async_copy(
            tab.at[:, pl.ds(base, CH)], staged.at[slot], sem_in).wait()
        gctr = extract(staged.at[slot], base, CH, gctr)

        @pl.when(i + 2 < nmine)
        def _():
            fire_chunk(i + 2, slot)

        return gctr

    gctr = lax.fori_loop(0, nmine, chunk_loop, 0)

    gctr2 = gctr
    if tail:
        def tail_extract():
            t_base = n_full * CH
            pltpu.sync_copy(tail_tab, tail_staged)
            return extract(tail_staged, t_base, tail, gctr)

        gctr2 = lax.cond(wid == tail_owner, tail_extract, lambda: gctr)

    def drain(i, carry):
        pltpu.make_async_copy(ext.at[0], _dummy_rows(), sem_out).wait()
        return carry

    lax.fori_loop(0, jnp.minimum(gctr2, NRING), drain, 0)


@functools.lru_cache(maxsize=None)
def _make_stream_kernel(NU, NM):
    mesh = plsc.VectorSubcoreMesh(core_axis_name="c", subcore_axis_name="s")

    @functools.partial(
        pl.kernel,
        out_type=(jax.ShapeDtypeStruct((B + PAD, 128), jnp.float32),
                  jax.ShapeDtypeStruct((B + PAD, 128), jnp.float32)),
        mesh=mesh,
        scratch_types=[
            pltpu.VMEM((B,), jnp.int32),
            pltpu.VMEM((HIT_CAP + L,), jnp.int32),
            pltpu.VMEM((HIT_CAP + L,), jnp.int32),
            pltpu.VMEM((CHIT_CAP + L,), jnp.int32),
            pltpu.VMEM((CHIT_CAP + L,), jnp.int32),
            pltpu.VMEM((2, D, CH), jnp.float32),
            pltpu.VMEM((D, NU % CH), jnp.float32),
            pltpu.VMEM((D, NM % CH), jnp.float32),
            pltpu.VMEM((NRING, L, 128), jnp.float32),
            pltpu.VMEM((NRING, L), jnp.int32),
            pltpu.SemaphoreType.DMA,
            pltpu.SemaphoreType.DMA,
        ],
        compiler_params=pltpu.CompilerParams(
            needs_layout_passes=False, use_tc_tiling_on_sc=True),
    )
    def stream_kernel(uid_hbm, mid_hbm, ut_hbm, mt_hbm, ut_tail, mt_tail,
                      uvec_hbm, mvec_hbm,
                      ids_v, hitu, hitb, cu, cb, staged, ut_ts, mt_ts,
                      ext, sidx, sem_in, sem_out):
        wid = lax.axis_index("s") * NC + lax.axis_index("c")

        _gather_pass(ut_hbm, ut_tail, uid_hbm, uvec_hbm, ids_v, hitu, hitb,
                     cu, cb, staged, ut_ts, ext, sidx, sem_in, sem_out,
                     wid, NU)
        _gather_pass(mt_hbm, mt_tail, mid_hbm, mvec_hbm, ids_v, hitu, hitb,
                     cu, cb, staged, mt_ts, ext, sidx, sem_in, sem_out,
                     wid, NM)

    return stream_kernel


@functools.lru_cache(maxsize=None)
def _make_dot_kernel():
    mesh = plsc.VectorSubcoreMesh(core_axis_name="c", subcore_axis_name="s")
    b_per_w = B // NW
    ST = 128

    @functools.partial(
        pl.kernel,
        out_type=jax.ShapeDtypeStruct((B,), jnp.float32),
        mesh=mesh,
        scratch_types=[
            pltpu.VMEM((ST * 128,), jnp.float32),
            pltpu.VMEM((ST * 128,), jnp.float32),
            pltpu.VMEM((ST * 128,), jnp.float32),
            pltpu.VMEM((ST * 128,), jnp.float32),
            pltpu.VMEM((b_per_w,), jnp.float32),
            pltpu.SemaphoreType.DMA,
        ],
        compiler_params=pltpu.CompilerParams(
            needs_layout_passes=False, use_tc_tiling_on_sc=True),
    )
    def dot_kernel(uvec_hbm, mvec_hbm, out_hbm, su0, su1, sm0, sm1, outv,
                   sem):
        wid = lax.axis_index("s") * NC + lax.axis_index("c")
        base = wid * b_per_w
        lanes = lax.iota(jnp.int32, L)
        n_stages = b_per_w // ST
        su = (su0, su1)
        sm = (sm0, sm1)

        def fire(i, slot):
            pltpu.make_async_copy(
                uvec_hbm.at[pl.ds((base + i * ST) * 128, ST * 128)],
                su[slot], sem).start()
            pltpu.make_async_copy(
                mvec_hbm.at[pl.ds((base + i * ST) * 128, ST * 128)],
                sm[slot], sem).start()

        fire(0, 0)
        fire(1, 1)

        for i in range(n_stages):
            slot = i % 2
            pltpu.make_async_copy(
                uvec_hbm.at[pl.ds((base + i * ST) * 128, ST * 128)],
                su[slot], sem).wait()
            pltpu.make_async_copy(
                mvec_hbm.at[pl.ds((base + i * ST) * 128, ST * 128)],
                sm[slot], sem).wait()

            def group_body(g, carry2, i=i, slot=slot):
                fl = (g * L + lanes) * 128
                acc0 = jnp.zeros((L,), jnp.float32)
                acc1 = jnp.zeros((L,), jnp.float32)
                for d in range(D // 2):
                    uu0 = plsc.load_gather(su[slot], [fl + d])
                    mm0 = plsc.load_gather(sm[slot], [fl + d])
                    acc0 = acc0 + uu0 * mm0
                    uu1 = plsc.load_gather(su[slot], [fl + (d + D // 2)])
                    mm1 = plsc.load_gather(sm[slot], [fl + (d + D // 2)])
                    acc1 = acc1 + uu1 * mm1
                outv[pl.ds(i * ST + g * L, L)] = acc0 + acc1
                return carry2

            lax.fori_loop(0, ST // L, group_body, 0)
            if i + 2 < n_stages:
                fire(i + 2, slot)

        pltpu.sync_copy(outv, out_hbm.at[pl.ds(base, b_per_w)])

    return dot_kernel


def kernel(user_ids, movie_ids, user_table, movie_table):
    NU = user_table.shape[0]
    NM = movie_table.shape[0]
    uids = user_ids.astype(jnp.int32)
    mids = movie_ids.astype(jnp.int32)
    ut_tail = user_table[(NU // CH) * CH:].T
    mt_tail = movie_table[(NM // CH) * CH:].T
    k1 = _make_stream_kernel(NU, NM)
    uvec, mvec = k1(uids, mids, user_table.T, movie_table.T, ut_tail,
                    mt_tail)
    k2 = _make_dot_kernel()
    return k2(uvec.reshape(-1), mvec.reshape(-1))
